# baseline (speedup 1.0000x reference)
import functools

import jax
import jax.numpy as jnp
from jax import lax
from jax.experimental import pallas as pl
from jax.experimental.pallas import tpu as pltpu
from jax.experimental.pallas import tpu_sc as plsc

N = 10000
E = 320000
EL = 100000
D = 128
H = 256

NC = 2
NS = 16
L = 16
NW = NC * NS

N_PAD = 10240
R_BLK = 512
K = 128
C0 = 160
C1 = 0
T_CH = NS * (C0 + C1)
E_PAD = T_CH * K
EPT = E_PAD // NW
PER_EL = -(-EL // (NW * L)) * L
EL_PAD = NW * PER_EL

import functools as _ft


@_ft.cache
def _mesh():
    return plsc.VectorSubcoreMesh(
        core_axis_name="c", subcore_axis_name="s",
        num_cores=NC, num_subcores=NS)


def _sc_agg1_body(x_hbm, srcm, dstm, zrow, zvec, sums_out, cnts_out,
                  idx_s, idx_d, rows0, rows1, cnt_v, acc_sh,
                  sem_g, sem_s, sem_is, sem_id):
    cid = lax.axis_index("c")
    sid = lax.axis_index("s")
    wid = sid * NC + cid
    rpt = N_PAD // NS
    lo = sid * rpt
    ones16 = jnp.ones((L,), jnp.float32)
    rows = (rows0, rows1)

    def start_idx(ch, b):
        pltpu.async_copy(srcm.at[ch], idx_s.at[b], sem_is.at[b])
        pltpu.async_copy(dstm.at[ch], idx_d.at[b], sem_id.at[b])

    def wait_idx(b):
        pltpu.make_async_copy(srcm.at[0], idx_s.at[b], sem_is.at[b]).wait()
        pltpu.make_async_copy(dstm.at[0], idx_d.at[b], sem_id.at[b]).wait()

    def start_gather(bi, rb):
        pltpu.async_copy(x_hbm.at[idx_s.at[bi]], rows[rb], sem_g.at[rb])

    def wait_gather(rb):
        pltpu.make_async_copy(x_hbm.at[idx_s.at[0]], rows[rb],
                              sem_g.at[rb]).wait()

    def start_scatter(bi, rb):
        pltpu.async_copy(rows[rb], acc_sh.at[idx_d.at[bi]], sem_s.at[rb],
                         add=True)

    def wait_scatter(rb):
        pltpu.make_async_copy(rows[rb], acc_sh.at[idx_d.at[0]],
                              sem_s.at[rb]).wait()

    pltpu.sync_copy(zrow.at[pl.ds(lo, rpt)], acc_sh.at[pl.ds(lo, rpt)])
    pltpu.sync_copy(zvec, cnt_v)
    plsc.subcore_barrier()

    def run_pipe(off, n):
        for b in range(4):
            start_idx(off + b, b)
        wait_idx(0)
        start_gather(0, 0)
        wait_idx(1)
        start_gather(1, 1)

        def body(i, carry):
            for b in range(4):
                jj = i * 4 + b
                rb = b % 2
                wait_gather(rb)
                start_scatter(b, rb)
                for ll in range(K // L):
                    dv = idx_d[b, pl.ds(ll * L, L)]
                    plsc.addupdate_scatter(cnt_v, [dv], ones16)
                wait_scatter(rb)

                @pl.when(jj + 2 < n)
                def _():
                    wait_idx((b + 2) % 4)
                    start_gather((b + 2) % 4, rb)

                @pl.when(jj + 4 < n)
                def _():
                    start_idx(off + jj + 4, b)
            return carry

        lax.fori_loop(0, n // 4, body, 0)

    @pl.when(cid == 0)
    def _():
        run_pipe(sid * C0, C0)

    if C1:
        @pl.when(cid == 1)
        def _():
            run_pipe(NS * C0 + sid * C1, C1)

    plsc.subcore_barrier()
    pltpu.sync_copy(acc_sh.at[pl.ds(lo, rpt)], sums_out.at[cid, pl.ds(lo, rpt)])
    pltpu.sync_copy(cnt_v, cnts_out.at[wid])


@_ft.cache
def _sc_agg1_k():
  return functools.partial(
    pl.kernel,
    out_type=(jax.ShapeDtypeStruct((NC, N_PAD, D), jnp.float32),
              jax.ShapeDtypeStruct((NW, N_PAD), jnp.float32)),
    mesh=_mesh(),
    compiler_params=pltpu.CompilerParams(needs_layout_passes=False),
    scratch_types=[
        pltpu.VMEM((4, K), jnp.int32),
        pltpu.VMEM((4, K), jnp.int32),
        pltpu.VMEM((K, D), jnp.float32),
        pltpu.VMEM((K, D), jnp.float32),
        pltpu.VMEM((N_PAD,), jnp.float32),
        pltpu.VMEM_SHARED((N_PAD, D), jnp.float32),
        pltpu.SemaphoreType.DMA((2,)),
        pltpu.SemaphoreType.DMA((2,)),
        pltpu.SemaphoreType.DMA((4,)),
        pltpu.SemaphoreType.DMA((4,)),
    ],
)(_sc_agg1_body)


def _sc_agg2_body(tcols, srcf, dstf, zvec, acc2_out,
                  src_v, dst_v, ta_v, tb_v, acca_v, accb_v):
    cid = lax.axis_index("c")
    sid = lax.axis_index("s")
    wid = sid * NC + cid
    pltpu.sync_copy(zvec, acca_v)
    pltpu.sync_copy(zvec, accb_v)
    pltpu.sync_copy(tcols.at[2], ta_v)
    pltpu.sync_copy(tcols.at[3], tb_v)
    pltpu.sync_copy(srcf.at[wid], src_v)
    pltpu.sync_copy(dstf.at[wid], dst_v)

    def body(j, carry):
        off = j * L
        sv = src_v[pl.ds(off, L)]
        dv = dst_v[pl.ds(off, L)]
        va = plsc.load_gather(ta_v, [sv])
        vb = plsc.load_gather(tb_v, [sv])
        plsc.addupdate_scatter(acca_v, [dv], va)
        plsc.addupdate_scatter(accb_v, [dv], vb)
        return carry

    lax.fori_loop(0, EPT // L, body, 0)
    pltpu.sync_copy(acca_v, acc2_out.at[wid, 0])
    pltpu.sync_copy(accb_v, acc2_out.at[wid, 1])


@_ft.cache
def _sc_agg2_k():
  return functools.partial(
    pl.kernel,
    out_type=jax.ShapeDtypeStruct((NW, 2, N_PAD), jnp.float32),
    mesh=_mesh(),
    compiler_params=pltpu.CompilerParams(needs_layout_passes=False),
    scratch_types=[
        pltpu.VMEM((EPT,), jnp.int32),
        pltpu.VMEM((EPT,), jnp.int32),
        pltpu.VMEM((N_PAD,), jnp.float32),
        pltpu.VMEM((N_PAD,), jnp.float32),
        pltpu.VMEM((N_PAD,), jnp.float32),
        pltpu.VMEM((N_PAD,), jnp.float32),
    ],
)(_sc_agg2_body)


def _sc_dec_body(pab_hbm, am, bm, out_hbm, pa_v, pb_v, ai_v, bi_v, out_v):
    cid = lax.axis_index("c")
    sid = lax.axis_index("s")
    wid = sid * NC + cid
    base = wid * PER_EL
    pltpu.sync_copy(pab_hbm.at[0], pa_v)
    pltpu.sync_copy(pab_hbm.at[1], pb_v)
    pltpu.sync_copy(am.at[wid], ai_v)
    pltpu.sync_copy(bm.at[wid], bi_v)

    def body(j, carry):
        off = j * L
        ia = ai_v[pl.ds(off, L)]
        ib = bi_v[pl.ds(off, L)]
        va = plsc.load_gather(pa_v, [ia])
        vb = plsc.load_gather(pb_v, [ib])
        out_v[pl.ds(off, L)] = va + vb
        return carry

    lax.fori_loop(0, PER_EL // L, body, 0)
    pltpu.sync_copy(out_v, out_hbm.at[pl.ds(base, PER_EL)])


@_ft.cache
def _sc_dec_k():
  return functools.partial(
    pl.kernel,
    out_type=jax.ShapeDtypeStruct((EL_PAD,), jnp.float32),
    mesh=_mesh(),
    compiler_params=pltpu.CompilerParams(needs_layout_passes=False),
    scratch_types=[
        pltpu.VMEM((N_PAD,), jnp.float32),
        pltpu.VMEM((N_PAD,), jnp.float32),
        pltpu.VMEM((PER_EL,), jnp.int32),
        pltpu.VMEM((PER_EL,), jnp.int32),
        pltpu.VMEM((PER_EL,), jnp.float32),
    ],
)(_sc_dec_body)


def _tc_main_body(x_ref, s_ref, c_ref, wl1_ref, wr1_ref, b1_ref,
                  wl2_ref, wr2_ref, w1_ref, w2_ref, w3_ref, t_ref):
    s = s_ref[0] + s_ref[1]
    cnt = jnp.sum(c_ref[...], axis=0)[:, None]
    mean = s / jnp.maximum(cnt, 1.0)
    z1 = jnp.maximum(
        jnp.dot(mean, wl1_ref[...], preferred_element_type=jnp.float32, precision=jax.lax.Precision.HIGHEST)
        + jnp.dot(x_ref[...], wr1_ref[...], preferred_element_type=jnp.float32, precision=jax.lax.Precision.HIGHEST)
        + b1_ref[...], 0.0)
    w23 = jnp.dot(w2_ref[...], w3_ref[...], preferred_element_type=jnp.float32, precision=jax.lax.Precision.HIGHEST)
    wfull = jnp.dot(w1_ref[...], w23, preferred_element_type=jnp.float32, precision=jax.lax.Precision.HIGHEST)
    wa = wfull[:H]
    wb = wfull[H:]
    v = jnp.concatenate(
        [jnp.dot(wr2_ref[...], wa, preferred_element_type=jnp.float32, precision=jax.lax.Precision.HIGHEST),
         jnp.dot(wr2_ref[...], wb, preferred_element_type=jnp.float32, precision=jax.lax.Precision.HIGHEST),
         jnp.dot(wl2_ref[...], wa, preferred_element_type=jnp.float32, precision=jax.lax.Precision.HIGHEST),
         jnp.dot(wl2_ref[...], wb, preferred_element_type=jnp.float32, precision=jax.lax.Precision.HIGHEST)],
        axis=1)
    t_ref[...] = jnp.dot(z1, v, preferred_element_type=jnp.float32, precision=jax.lax.Precision.HIGHEST).T


def _tc_main(x_pad, sums, cnts, Wl1, Wr1, b1, Wl2, Wr2, W1, W2, W3):
    grid = (N_PAD // R_BLK,)
    return pl.pallas_call(
        _tc_main_body,
        grid=grid,
        in_specs=[
            pl.BlockSpec((R_BLK, D), lambda i: (i, 0)),
            pl.BlockSpec((NC, R_BLK, D), lambda i: (0, i, 0)),
            pl.BlockSpec((NW, R_BLK), lambda i: (0, i)),
            pl.BlockSpec((D, H), lambda i: (0, 0)),
            pl.BlockSpec((D, H), lambda i: (0, 0)),
            pl.BlockSpec((1, H), lambda i: (0, 0)),
            pl.BlockSpec((H, H), lambda i: (0, 0)),
            pl.BlockSpec((H, H), lambda i: (0, 0)),
            pl.BlockSpec((2 * H, H), lambda i: (0, 0)),
            pl.BlockSpec((H, H), lambda i: (0, 0)),
            pl.BlockSpec((H, 1), lambda i: (0, 0)),
        ],
        out_specs=pl.BlockSpec((4, R_BLK), lambda i: (0, i)),
        out_shape=jax.ShapeDtypeStruct((4, N_PAD), jnp.float32),
    )(x_pad, sums, cnts, Wl1, Wr1, b1, Wl2, Wr2, W1, W2, W3)


def _tc_fin_body(acc2_ref, cnts_ref, t_ref, w1_ref, w2_ref, w3_ref,
                 b2_ref, c1_ref, c2_ref, c3_ref, pab_ref):
    w23 = jnp.dot(w2_ref[...], w3_ref[...], preferred_element_type=jnp.float32, precision=jax.lax.Precision.HIGHEST)
    wfull = jnp.dot(w1_ref[...], w23, preferred_element_type=jnp.float32, precision=jax.lax.Precision.HIGHEST)
    wa = wfull[:H, 0]
    wb = wfull[H:, 0]
    cfull = (jnp.sum(c1_ref[0] * w23[:, 0]) + jnp.sum(c2_ref[0] * w3_ref[:, 0])
             + c3_ref[0, 0])
    ba = jnp.sum(b2_ref[0] * wa)
    bb = jnp.sum(b2_ref[0] * wb)
    cnt = jnp.maximum(jnp.sum(cnts_ref[...], axis=0), 1.0)
    m_a = jnp.sum(acc2_ref[:, 0, :], axis=0) / cnt
    m_b = jnp.sum(acc2_ref[:, 1, :], axis=0) / cnt
    pab_ref[0, :] = m_a + t_ref[0, :] + ba + cfull
    pab_ref[1, :] = m_b + t_ref[1, :] + bb


def _tc_fin(acc2, cnts, tcols, W1, W2, W3, b2, c1, c2, c3):
    return pl.pallas_call(
        _tc_fin_body,
        out_shape=jax.ShapeDtypeStruct((2, N_PAD), jnp.float32),
    )(acc2, cnts, tcols, W1, W2, W3, b2, c1, c2, c3)


def kernel(x, edge_index, edge_label_index, Wl1, Wr1, b1, Wl2, Wr2, b2,
           W1, c1, W2, c2, W3, c3):
    f32 = jnp.float32
    i32 = jnp.int32
    src = edge_index[0].astype(i32)
    dst = edge_index[1].astype(i32)
    src_pad = jnp.concatenate([src, jnp.zeros((E_PAD - E,), i32)])
    dst_pad = jnp.concatenate([dst, jnp.full((E_PAD - E,), N, i32)])
    srcm = src_pad.reshape(T_CH, K)
    dstm = dst_pad.reshape(T_CH, K)
    srcf = src_pad.reshape(NW, EPT)
    dstf = dst_pad.reshape(NW, EPT)
    am = jnp.concatenate(
        [edge_label_index[0].astype(i32),
         jnp.zeros((EL_PAD - EL,), i32)]).reshape(NW, PER_EL)
    bm = jnp.concatenate(
        [edge_label_index[1].astype(i32),
         jnp.zeros((EL_PAD - EL,), i32)]).reshape(NW, PER_EL)
    x_pad = jnp.zeros((N_PAD, D), f32).at[:N].set(x)
    zrow = jnp.zeros((N_PAD, D), f32)
    zvec = jnp.zeros((N_PAD,), f32)

    sums, cnts = _sc_agg1_k()(x_pad, srcm, dstm, zrow, zvec)
    tcols = _tc_main(x_pad, sums, cnts, Wl1, Wr1, b1.reshape(1, H),
                     Wl2, Wr2, W1, W2, W3)
    acc2 = _sc_agg2_k()(tcols, srcf, dstf, zvec)
    pab = _tc_fin(acc2, cnts, tcols, W1, W2, W3, b2.reshape(1, H),
                  c1.reshape(1, H), c2.reshape(1, H), c3.reshape(1, 1))
    outf = _sc_dec_k()(pab, am, bm)
    return outf[:EL].reshape(EL, 1)


# speedup vs baseline: 1.7707x; 1.7707x over previous
import functools

import jax
import jax.numpy as jnp
from jax import lax
from jax.experimental import pallas as pl
from jax.experimental.pallas import tpu as pltpu
from jax.experimental.pallas import tpu_sc as plsc

N = 10000
E = 320000
EL = 100000
D = 128
H = 256

NC = 2
NS = 16
L = 16
NW = NC * NS

N_PAD = 10240
R_BLK = 512
K = 128
C0 = 160
C1 = 0
T_CH = NS * (C0 + C1)
E_PAD = T_CH * K
EPT = E_PAD // NW
PER_EL = -(-EL // (NW * L)) * L
EL_PAD = NW * PER_EL

import functools as _ft


@_ft.cache
def _mesh():
    return plsc.VectorSubcoreMesh(
        core_axis_name="c", subcore_axis_name="s",
        num_cores=NC, num_subcores=NS)


def _sc_agg1_body(x_hbm, srcm, dstm, zrow, zvec, sums_out, cnts_out,
                  idx_s, idx_d, rows0, rows1, cnt_v, acc_sh,
                  sem_g, sem_s, sem_is, sem_id):
    cid = lax.axis_index("c")
    sid = lax.axis_index("s")
    wid = sid * NC + cid
    rpt = N_PAD // NS
    lo = sid * rpt
    ones16 = jnp.ones((L,), jnp.float32)
    rows = (rows0, rows1)

    def start_idx(ch, b):
        pltpu.async_copy(srcm.at[ch], idx_s.at[b], sem_is.at[b])
        pltpu.async_copy(dstm.at[ch], idx_d.at[b], sem_id.at[b])

    def wait_idx(b):
        pltpu.make_async_copy(srcm.at[0], idx_s.at[b], sem_is.at[b]).wait()
        pltpu.make_async_copy(dstm.at[0], idx_d.at[b], sem_id.at[b]).wait()

    def start_gather(bi, rb):
        pltpu.async_copy(x_hbm.at[idx_s.at[bi]], rows[rb], sem_g.at[rb])

    def wait_gather(rb):
        pltpu.make_async_copy(x_hbm.at[idx_s.at[0]], rows[rb],
                              sem_g.at[rb]).wait()

    def start_scatter(bi, rb):
        pltpu.async_copy(rows[rb], acc_sh.at[idx_d.at[bi]], sem_s.at[rb],
                         add=True)

    def wait_scatter(rb):
        pltpu.make_async_copy(rows[rb], acc_sh.at[idx_d.at[0]],
                              sem_s.at[rb]).wait()

    pltpu.sync_copy(zrow.at[pl.ds(lo, rpt)], acc_sh.at[pl.ds(lo, rpt)])
    pltpu.sync_copy(zvec, cnt_v)
    plsc.subcore_barrier()

    def run_pipe(off, n):
        for b in range(4):
            start_idx(off + b, b)
        wait_idx(0)
        start_gather(0, 0)
        wait_idx(1)
        start_gather(1, 1)

        def body(i, carry):
            for b in range(4):
                jj = i * 4 + b
                rb = b % 2
                wait_gather(rb)
                start_scatter(b, rb)
                for ll in range(K // L):
                    dv = idx_d[b, pl.ds(ll * L, L)]
                    plsc.addupdate_scatter(cnt_v, [dv], ones16)
                wait_scatter(rb)

                @pl.when(jj + 2 < n)
                def _():
                    wait_idx((b + 2) % 4)
                    start_gather((b + 2) % 4, rb)

                @pl.when(jj + 4 < n)
                def _():
                    start_idx(off + jj + 4, b)
            return carry

        lax.fori_loop(0, n // 4, body, 0)

    @pl.when(cid == 0)
    def _():
        run_pipe(sid * C0, C0)

    if C1:
        @pl.when(cid == 1)
        def _():
            run_pipe(NS * C0 + sid * C1, C1)

    plsc.subcore_barrier()
    pltpu.sync_copy(acc_sh.at[pl.ds(lo, rpt)], sums_out.at[cid, pl.ds(lo, rpt)])
    pltpu.sync_copy(cnt_v, cnts_out.at[wid])


@_ft.cache
def _sc_agg1_k():
  return functools.partial(
    pl.kernel,
    out_type=(jax.ShapeDtypeStruct((NC, N_PAD, D), jnp.float32),
              jax.ShapeDtypeStruct((NW, N_PAD), jnp.float32)),
    mesh=_mesh(),
    compiler_params=pltpu.CompilerParams(needs_layout_passes=False),
    scratch_types=[
        pltpu.VMEM((4, K), jnp.int32),
        pltpu.VMEM((4, K), jnp.int32),
        pltpu.VMEM((K, D), jnp.float32),
        pltpu.VMEM((K, D), jnp.float32),
        pltpu.VMEM((N_PAD,), jnp.float32),
        pltpu.VMEM_SHARED((N_PAD, D), jnp.float32),
        pltpu.SemaphoreType.DMA((2,)),
        pltpu.SemaphoreType.DMA((2,)),
        pltpu.SemaphoreType.DMA((4,)),
        pltpu.SemaphoreType.DMA((4,)),
    ],
)(_sc_agg1_body)


def _sc_agg2_body(tcols, srcf, dstf, zvec, acc2_out,
                  src_v, dst_v, ta_v, tb_v, acca_v, accb_v):
    cid = lax.axis_index("c")
    sid = lax.axis_index("s")
    wid = sid * NC + cid
    pltpu.sync_copy(zvec, acca_v)
    pltpu.sync_copy(zvec, accb_v)
    pltpu.sync_copy(tcols.at[2], ta_v)
    pltpu.sync_copy(tcols.at[3], tb_v)
    pltpu.sync_copy(srcf.at[wid], src_v)
    pltpu.sync_copy(dstf.at[wid], dst_v)

    def body(j, carry):
        off = j * L
        sv = src_v[pl.ds(off, L)]
        dv = dst_v[pl.ds(off, L)]
        va = plsc.load_gather(ta_v, [sv])
        vb = plsc.load_gather(tb_v, [sv])
        plsc.addupdate_scatter(acca_v, [dv], va)
        plsc.addupdate_scatter(accb_v, [dv], vb)
        return carry

    lax.fori_loop(0, EPT // L, body, 0)
    pltpu.sync_copy(acca_v, acc2_out.at[wid, 0])
    pltpu.sync_copy(accb_v, acc2_out.at[wid, 1])


@_ft.cache
def _sc_agg2_k():
  return functools.partial(
    pl.kernel,
    out_type=jax.ShapeDtypeStruct((NW, 2, N_PAD), jnp.float32),
    mesh=_mesh(),
    compiler_params=pltpu.CompilerParams(needs_layout_passes=False),
    scratch_types=[
        pltpu.VMEM((EPT,), jnp.int32),
        pltpu.VMEM((EPT,), jnp.int32),
        pltpu.VMEM((N_PAD,), jnp.float32),
        pltpu.VMEM((N_PAD,), jnp.float32),
        pltpu.VMEM((N_PAD,), jnp.float32),
        pltpu.VMEM((N_PAD,), jnp.float32),
    ],
)(_sc_agg2_body)


def _sc_dec_body(pab_hbm, am, bm, out_hbm, pa_v, pb_v, ai_v, bi_v, out_v):
    cid = lax.axis_index("c")
    sid = lax.axis_index("s")
    wid = sid * NC + cid
    base = wid * PER_EL
    pltpu.sync_copy(pab_hbm.at[0], pa_v)
    pltpu.sync_copy(pab_hbm.at[1], pb_v)
    pltpu.sync_copy(am.at[wid], ai_v)
    pltpu.sync_copy(bm.at[wid], bi_v)

    def body(j, carry):
        off = j * L
        ia = ai_v[pl.ds(off, L)]
        ib = bi_v[pl.ds(off, L)]
        va = plsc.load_gather(pa_v, [ia])
        vb = plsc.load_gather(pb_v, [ib])
        out_v[pl.ds(off, L)] = va + vb
        return carry

    lax.fori_loop(0, PER_EL // L, body, 0)
    pltpu.sync_copy(out_v, out_hbm.at[pl.ds(base, PER_EL)])


@_ft.cache
def _sc_dec_k():
  return functools.partial(
    pl.kernel,
    out_type=jax.ShapeDtypeStruct((EL_PAD,), jnp.float32),
    mesh=_mesh(),
    compiler_params=pltpu.CompilerParams(needs_layout_passes=False),
    scratch_types=[
        pltpu.VMEM((N_PAD,), jnp.float32),
        pltpu.VMEM((N_PAD,), jnp.float32),
        pltpu.VMEM((PER_EL,), jnp.int32),
        pltpu.VMEM((PER_EL,), jnp.int32),
        pltpu.VMEM((PER_EL,), jnp.float32),
    ],
)(_sc_dec_body)


def _tc_main_body(x_ref, s_ref, c_ref, wl1_ref, wr1_ref, b1_ref,
                  wl2_ref, wr2_ref, w1_ref, w2_ref, w3_ref, t_ref):
    s = s_ref[0] + s_ref[1]
    cnt = jnp.sum(c_ref[...], axis=0)[:, None]
    mean = s / jnp.maximum(cnt, 1.0)
    z1 = jnp.maximum(
        jnp.dot(mean, wl1_ref[...], preferred_element_type=jnp.float32, precision=jax.lax.Precision.HIGHEST)
        + jnp.dot(x_ref[...], wr1_ref[...], preferred_element_type=jnp.float32, precision=jax.lax.Precision.HIGHEST)
        + b1_ref[...], 0.0)
    w23 = jnp.dot(w2_ref[...], w3_ref[...], preferred_element_type=jnp.float32, precision=jax.lax.Precision.HIGHEST)
    wfull = jnp.dot(w1_ref[...], w23, preferred_element_type=jnp.float32, precision=jax.lax.Precision.HIGHEST)
    wa = wfull[:H]
    wb = wfull[H:]
    v = jnp.concatenate(
        [jnp.dot(wr2_ref[...], wa, preferred_element_type=jnp.float32, precision=jax.lax.Precision.HIGHEST),
         jnp.dot(wr2_ref[...], wb, preferred_element_type=jnp.float32, precision=jax.lax.Precision.HIGHEST),
         jnp.dot(wl2_ref[...], wa, preferred_element_type=jnp.float32, precision=jax.lax.Precision.HIGHEST),
         jnp.dot(wl2_ref[...], wb, preferred_element_type=jnp.float32, precision=jax.lax.Precision.HIGHEST)],
        axis=1)
    t_ref[...] = jnp.dot(z1, v, preferred_element_type=jnp.float32, precision=jax.lax.Precision.HIGHEST).T


def _tc_main(x_pad, sums, cnts, Wl1, Wr1, b1, Wl2, Wr2, W1, W2, W3):
    grid = (N_PAD // R_BLK,)
    return pl.pallas_call(
        _tc_main_body,
        grid=grid,
        in_specs=[
            pl.BlockSpec((R_BLK, D), lambda i: (i, 0)),
            pl.BlockSpec((NC, R_BLK, D), lambda i: (0, i, 0)),
            pl.BlockSpec((NW, R_BLK), lambda i: (0, i)),
            pl.BlockSpec((D, H), lambda i: (0, 0)),
            pl.BlockSpec((D, H), lambda i: (0, 0)),
            pl.BlockSpec((1, H), lambda i: (0, 0)),
            pl.BlockSpec((H, H), lambda i: (0, 0)),
            pl.BlockSpec((H, H), lambda i: (0, 0)),
            pl.BlockSpec((2 * H, H), lambda i: (0, 0)),
            pl.BlockSpec((H, H), lambda i: (0, 0)),
            pl.BlockSpec((H, 1), lambda i: (0, 0)),
        ],
        out_specs=pl.BlockSpec((4, R_BLK), lambda i: (0, i)),
        out_shape=jax.ShapeDtypeStruct((4, N_PAD), jnp.float32),
    )(x_pad, sums, cnts, Wl1, Wr1, b1, Wl2, Wr2, W1, W2, W3)


def _tc_fin_body(acc2_ref, cnts_ref, t_ref, w1_ref, w2_ref, w3_ref,
                 b2_ref, c1_ref, c2_ref, c3_ref, pab_ref):
    w23 = jnp.dot(w2_ref[...], w3_ref[...], preferred_element_type=jnp.float32, precision=jax.lax.Precision.HIGHEST)
    wfull = jnp.dot(w1_ref[...], w23, preferred_element_type=jnp.float32, precision=jax.lax.Precision.HIGHEST)
    wa = wfull[:H, 0]
    wb = wfull[H:, 0]
    cfull = (jnp.sum(c1_ref[0] * w23[:, 0]) + jnp.sum(c2_ref[0] * w3_ref[:, 0])
             + c3_ref[0, 0])
    ba = jnp.sum(b2_ref[0] * wa)
    bb = jnp.sum(b2_ref[0] * wb)
    cnt = jnp.maximum(jnp.sum(cnts_ref[...], axis=0), 1.0)
    m_a = jnp.sum(acc2_ref[:, 0, :], axis=0) / cnt
    m_b = jnp.sum(acc2_ref[:, 1, :], axis=0) / cnt
    pab_ref[0, :] = m_a + t_ref[0, :] + ba + cfull
    pab_ref[1, :] = m_b + t_ref[1, :] + bb


def _tc_fin(acc2, cnts, tcols, W1, W2, W3, b2, c1, c2, c3):
    return pl.pallas_call(
        _tc_fin_body,
        out_shape=jax.ShapeDtypeStruct((2, N_PAD), jnp.float32),
    )(acc2, cnts, tcols, W1, W2, W3, b2, c1, c2, c3)


def kernel(x, edge_index, edge_label_index, Wl1, Wr1, b1, Wl2, Wr2, b2,
           W1, c1, W2, c2, W3, c3):
    f32 = jnp.float32
    i32 = jnp.int32
    src = edge_index[0].astype(i32)
    dst = edge_index[1].astype(i32)
    pad_src = jnp.arange(E_PAD - E, dtype=i32) % N
    pad_dst = N + jnp.arange(E_PAD - E, dtype=i32) % (N_PAD - N)
    src_pad = jnp.concatenate([src, pad_src])
    dst_pad = jnp.concatenate([dst, pad_dst])
    srcm = src_pad.reshape(T_CH, K)
    dstm = dst_pad.reshape(T_CH, K)
    srcf = src_pad.reshape(NW, EPT)
    dstf = dst_pad.reshape(NW, EPT)
    am = jnp.concatenate(
        [edge_label_index[0].astype(i32),
         jnp.zeros((EL_PAD - EL,), i32)]).reshape(NW, PER_EL)
    bm = jnp.concatenate(
        [edge_label_index[1].astype(i32),
         jnp.zeros((EL_PAD - EL,), i32)]).reshape(NW, PER_EL)
    x_pad = jnp.zeros((N_PAD, D), f32).at[:N].set(x)
    zrow = jnp.zeros((N_PAD, D), f32)
    zvec = jnp.zeros((N_PAD,), f32)

    sums, cnts = _sc_agg1_k()(x_pad, srcm, dstm, zrow, zvec)
    tcols = _tc_main(x_pad, sums, cnts, Wl1, Wr1, b1.reshape(1, H),
                     Wl2, Wr2, W1, W2, W3)
    acc2 = _sc_agg2_k()(tcols, srcf, dstf, zvec)
    pab = _tc_fin(acc2, cnts, tcols, W1, W2, W3, b2.reshape(1, H),
                  c1.reshape(1, H), c2.reshape(1, H), c3.reshape(1, 1))
    outf = _sc_dec_k()(pab, am, bm)
    return outf[:EL].reshape(EL, 1)


# speedup vs baseline: 2.3314x; 1.3167x over previous
import functools

import jax
import jax.numpy as jnp
from jax import lax
from jax.experimental import pallas as pl
from jax.experimental.pallas import tpu as pltpu
from jax.experimental.pallas import tpu_sc as plsc

N = 10000
E = 320000
EL = 100000
D = 128
H = 256

NC = 2
NS = 16
L = 16
NW = NC * NS

N_PAD = 10240
R_BLK = 512
K = 128
C0 = 80
C1 = 80
T_CH = NS * (C0 + C1)
E_PAD = T_CH * K
EPT = E_PAD // NW
PER_EL = -(-EL // (NW * L)) * L
EL_PAD = NW * PER_EL

import functools as _ft


@_ft.cache
def _mesh():
    return plsc.VectorSubcoreMesh(
        core_axis_name="c", subcore_axis_name="s",
        num_cores=NC, num_subcores=NS)


def _sc_agg1_body(x_hbm, srcm, dstm, zrow, zvec, sums_out, cnts_out,
                  idx_s, idx_d, rows0, rows1, cnt_v, acc_sh,
                  sem_g, sem_s, sem_is, sem_id):
    cid = lax.axis_index("c")
    sid = lax.axis_index("s")
    wid = sid * NC + cid
    rpt = N_PAD // NS
    lo = sid * rpt
    ones16 = jnp.ones((L,), jnp.float32)
    rows = (rows0, rows1)

    def start_idx(ch, b):
        pltpu.async_copy(srcm.at[ch], idx_s.at[b], sem_is.at[b])
        pltpu.async_copy(dstm.at[ch], idx_d.at[b], sem_id.at[b])

    def wait_idx(b):
        pltpu.make_async_copy(srcm.at[0], idx_s.at[b], sem_is.at[b]).wait()
        pltpu.make_async_copy(dstm.at[0], idx_d.at[b], sem_id.at[b]).wait()

    def start_gather(bi, rb):
        pltpu.async_copy(x_hbm.at[idx_s.at[bi]], rows[rb], sem_g.at[rb])

    def wait_gather(rb):
        pltpu.make_async_copy(x_hbm.at[idx_s.at[0]], rows[rb],
                              sem_g.at[rb]).wait()

    def start_scatter(bi, rb):
        pltpu.async_copy(rows[rb], acc_sh.at[idx_d.at[bi]], sem_s.at[rb],
                         add=True)

    def wait_scatter(rb):
        pltpu.make_async_copy(rows[rb], acc_sh.at[idx_d.at[0]],
                              sem_s.at[rb]).wait()

    pltpu.sync_copy(zrow.at[pl.ds(lo, rpt)], acc_sh.at[pl.ds(lo, rpt)])
    pltpu.sync_copy(zvec, cnt_v)
    plsc.subcore_barrier()

    def run_pipe(off, n):
        for b in range(4):
            start_idx(off + b, b)
        wait_idx(0)
        start_gather(0, 0)
        wait_idx(1)
        start_gather(1, 1)

        def body(i, carry):
            for b in range(4):
                jj = i * 4 + b
                rb = b % 2
                wait_gather(rb)
                start_scatter(b, rb)
                for ll in range(K // L):
                    dv = idx_d[b, pl.ds(ll * L, L)]
                    plsc.addupdate_scatter(cnt_v, [dv], ones16)
                wait_scatter(rb)

                @pl.when(jj + 2 < n)
                def _():
                    wait_idx((b + 2) % 4)
                    start_gather((b + 2) % 4, rb)

                @pl.when(jj + 4 < n)
                def _():
                    start_idx(off + jj + 4, b)
            return carry

        lax.fori_loop(0, n // 4, body, 0)

    @pl.when(cid == 0)
    def _():
        run_pipe(sid * C0, C0)

    if C1:
        @pl.when(cid == 1)
        def _():
            run_pipe(NS * C0 + sid * C1, C1)

    plsc.subcore_barrier()
    pltpu.sync_copy(acc_sh.at[pl.ds(lo, rpt)], sums_out.at[cid, pl.ds(lo, rpt)])
    pltpu.sync_copy(cnt_v, cnts_out.at[wid])


@_ft.cache
def _sc_agg1_k():
  return functools.partial(
    pl.kernel,
    out_type=(jax.ShapeDtypeStruct((NC, N_PAD, D), jnp.float32),
              jax.ShapeDtypeStruct((NW, N_PAD), jnp.float32)),
    mesh=_mesh(),
    compiler_params=pltpu.CompilerParams(needs_layout_passes=False),
    scratch_types=[
        pltpu.VMEM((4, K), jnp.int32),
        pltpu.VMEM((4, K), jnp.int32),
        pltpu.VMEM((K, D), jnp.float32),
        pltpu.VMEM((K, D), jnp.float32),
        pltpu.VMEM((N_PAD,), jnp.float32),
        pltpu.VMEM_SHARED((N_PAD, D), jnp.float32),
        pltpu.SemaphoreType.DMA((2,)),
        pltpu.SemaphoreType.DMA((2,)),
        pltpu.SemaphoreType.DMA((4,)),
        pltpu.SemaphoreType.DMA((4,)),
    ],
)(_sc_agg1_body)


def _sc_agg2_body(tcols, srcf, dstf, zvec, acc2_out,
                  src_v, dst_v, ta_v, tb_v, acca_v, accb_v):
    cid = lax.axis_index("c")
    sid = lax.axis_index("s")
    wid = sid * NC + cid
    pltpu.sync_copy(zvec, acca_v)
    pltpu.sync_copy(zvec, accb_v)
    pltpu.sync_copy(tcols.at[2], ta_v)
    pltpu.sync_copy(tcols.at[3], tb_v)
    pltpu.sync_copy(srcf.at[wid], src_v)
    pltpu.sync_copy(dstf.at[wid], dst_v)

    def body(j, carry):
        off = j * L
        sv = src_v[pl.ds(off, L)]
        dv = dst_v[pl.ds(off, L)]
        va = plsc.load_gather(ta_v, [sv])
        vb = plsc.load_gather(tb_v, [sv])
        plsc.addupdate_scatter(acca_v, [dv], va)
        plsc.addupdate_scatter(accb_v, [dv], vb)
        return carry

    lax.fori_loop(0, EPT // L, body, 0)
    pltpu.sync_copy(acca_v, acc2_out.at[wid, 0])
    pltpu.sync_copy(accb_v, acc2_out.at[wid, 1])


@_ft.cache
def _sc_agg2_k():
  return functools.partial(
    pl.kernel,
    out_type=jax.ShapeDtypeStruct((NW, 2, N_PAD), jnp.float32),
    mesh=_mesh(),
    compiler_params=pltpu.CompilerParams(needs_layout_passes=False),
    scratch_types=[
        pltpu.VMEM((EPT,), jnp.int32),
        pltpu.VMEM((EPT,), jnp.int32),
        pltpu.VMEM((N_PAD,), jnp.float32),
        pltpu.VMEM((N_PAD,), jnp.float32),
        pltpu.VMEM((N_PAD,), jnp.float32),
        pltpu.VMEM((N_PAD,), jnp.float32),
    ],
)(_sc_agg2_body)


def _sc_dec_body(pab_hbm, am, bm, out_hbm, pa_v, pb_v, ai_v, bi_v, out_v):
    cid = lax.axis_index("c")
    sid = lax.axis_index("s")
    wid = sid * NC + cid
    base = wid * PER_EL
    pltpu.sync_copy(pab_hbm.at[0], pa_v)
    pltpu.sync_copy(pab_hbm.at[1], pb_v)
    pltpu.sync_copy(am.at[wid], ai_v)
    pltpu.sync_copy(bm.at[wid], bi_v)

    def body(j, carry):
        off = j * L
        ia = ai_v[pl.ds(off, L)]
        ib = bi_v[pl.ds(off, L)]
        va = plsc.load_gather(pa_v, [ia])
        vb = plsc.load_gather(pb_v, [ib])
        out_v[pl.ds(off, L)] = va + vb
        return carry

    lax.fori_loop(0, PER_EL // L, body, 0)
    pltpu.sync_copy(out_v, out_hbm.at[pl.ds(base, PER_EL)])


@_ft.cache
def _sc_dec_k():
  return functools.partial(
    pl.kernel,
    out_type=jax.ShapeDtypeStruct((EL_PAD,), jnp.float32),
    mesh=_mesh(),
    compiler_params=pltpu.CompilerParams(needs_layout_passes=False),
    scratch_types=[
        pltpu.VMEM((N_PAD,), jnp.float32),
        pltpu.VMEM((N_PAD,), jnp.float32),
        pltpu.VMEM((PER_EL,), jnp.int32),
        pltpu.VMEM((PER_EL,), jnp.int32),
        pltpu.VMEM((PER_EL,), jnp.float32),
    ],
)(_sc_dec_body)


def _tc_main_body(x_ref, s_ref, c_ref, wl1_ref, wr1_ref, b1_ref,
                  wl2_ref, wr2_ref, w1_ref, w2_ref, w3_ref, t_ref):
    s = s_ref[0] + s_ref[1]
    cnt = jnp.sum(c_ref[...], axis=0)[:, None]
    mean = s / jnp.maximum(cnt, 1.0)
    z1 = jnp.maximum(
        jnp.dot(mean, wl1_ref[...], preferred_element_type=jnp.float32, precision=jax.lax.Precision.HIGHEST)
        + jnp.dot(x_ref[...], wr1_ref[...], preferred_element_type=jnp.float32, precision=jax.lax.Precision.HIGHEST)
        + b1_ref[...], 0.0)
    w23 = jnp.dot(w2_ref[...], w3_ref[...], preferred_element_type=jnp.float32, precision=jax.lax.Precision.HIGHEST)
    wfull = jnp.dot(w1_ref[...], w23, preferred_element_type=jnp.float32, precision=jax.lax.Precision.HIGHEST)
    wa = wfull[:H]
    wb = wfull[H:]
    v = jnp.concatenate(
        [jnp.dot(wr2_ref[...], wa, preferred_element_type=jnp.float32, precision=jax.lax.Precision.HIGHEST),
         jnp.dot(wr2_ref[...], wb, preferred_element_type=jnp.float32, precision=jax.lax.Precision.HIGHEST),
         jnp.dot(wl2_ref[...], wa, preferred_element_type=jnp.float32, precision=jax.lax.Precision.HIGHEST),
         jnp.dot(wl2_ref[...], wb, preferred_element_type=jnp.float32, precision=jax.lax.Precision.HIGHEST)],
        axis=1)
    t_ref[...] = jnp.dot(z1, v, preferred_element_type=jnp.float32, precision=jax.lax.Precision.HIGHEST).T


def _tc_main(x_pad, sums, cnts, Wl1, Wr1, b1, Wl2, Wr2, W1, W2, W3):
    grid = (N_PAD // R_BLK,)
    return pl.pallas_call(
        _tc_main_body,
        grid=grid,
        in_specs=[
            pl.BlockSpec((R_BLK, D), lambda i: (i, 0)),
            pl.BlockSpec((NC, R_BLK, D), lambda i: (0, i, 0)),
            pl.BlockSpec((NW, R_BLK), lambda i: (0, i)),
            pl.BlockSpec((D, H), lambda i: (0, 0)),
            pl.BlockSpec((D, H), lambda i: (0, 0)),
            pl.BlockSpec((1, H), lambda i: (0, 0)),
            pl.BlockSpec((H, H), lambda i: (0, 0)),
            pl.BlockSpec((H, H), lambda i: (0, 0)),
            pl.BlockSpec((2 * H, H), lambda i: (0, 0)),
            pl.BlockSpec((H, H), lambda i: (0, 0)),
            pl.BlockSpec((H, 1), lambda i: (0, 0)),
        ],
        out_specs=pl.BlockSpec((4, R_BLK), lambda i: (0, i)),
        out_shape=jax.ShapeDtypeStruct((4, N_PAD), jnp.float32),
    )(x_pad, sums, cnts, Wl1, Wr1, b1, Wl2, Wr2, W1, W2, W3)


def _tc_fin_body(acc2_ref, cnts_ref, t_ref, w1_ref, w2_ref, w3_ref,
                 b2_ref, c1_ref, c2_ref, c3_ref, pab_ref):
    w23 = jnp.dot(w2_ref[...], w3_ref[...], preferred_element_type=jnp.float32, precision=jax.lax.Precision.HIGHEST)
    wfull = jnp.dot(w1_ref[...], w23, preferred_element_type=jnp.float32, precision=jax.lax.Precision.HIGHEST)
    wa = wfull[:H, 0]
    wb = wfull[H:, 0]
    cfull = (jnp.sum(c1_ref[0] * w23[:, 0]) + jnp.sum(c2_ref[0] * w3_ref[:, 0])
             + c3_ref[0, 0])
    ba = jnp.sum(b2_ref[0] * wa)
    bb = jnp.sum(b2_ref[0] * wb)
    cnt = jnp.maximum(jnp.sum(cnts_ref[...], axis=0), 1.0)
    m_a = jnp.sum(acc2_ref[:, 0, :], axis=0) / cnt
    m_b = jnp.sum(acc2_ref[:, 1, :], axis=0) / cnt
    pab_ref[0, :] = m_a + t_ref[0, :] + ba + cfull
    pab_ref[1, :] = m_b + t_ref[1, :] + bb


def _tc_fin(acc2, cnts, tcols, W1, W2, W3, b2, c1, c2, c3):
    return pl.pallas_call(
        _tc_fin_body,
        out_shape=jax.ShapeDtypeStruct((2, N_PAD), jnp.float32),
    )(acc2, cnts, tcols, W1, W2, W3, b2, c1, c2, c3)


def kernel(x, edge_index, edge_label_index, Wl1, Wr1, b1, Wl2, Wr2, b2,
           W1, c1, W2, c2, W3, c3):
    f32 = jnp.float32
    i32 = jnp.int32
    src = edge_index[0].astype(i32)
    dst = edge_index[1].astype(i32)
    pad_src = jnp.arange(E_PAD - E, dtype=i32) % N
    pad_dst = N + jnp.arange(E_PAD - E, dtype=i32) % (N_PAD - N)
    src_pad = jnp.concatenate([src, pad_src])
    dst_pad = jnp.concatenate([dst, pad_dst])
    srcm = src_pad.reshape(T_CH, K)
    dstm = dst_pad.reshape(T_CH, K)
    srcf = src_pad.reshape(NW, EPT)
    dstf = dst_pad.reshape(NW, EPT)
    am = jnp.concatenate(
        [edge_label_index[0].astype(i32),
         jnp.zeros((EL_PAD - EL,), i32)]).reshape(NW, PER_EL)
    bm = jnp.concatenate(
        [edge_label_index[1].astype(i32),
         jnp.zeros((EL_PAD - EL,), i32)]).reshape(NW, PER_EL)
    x_pad = jnp.zeros((N_PAD, D), f32).at[:N].set(x)
    zrow = jnp.zeros((N_PAD, D), f32)
    zvec = jnp.zeros((N_PAD,), f32)

    sums, cnts = _sc_agg1_k()(x_pad, srcm, dstm, zrow, zvec)
    tcols = _tc_main(x_pad, sums, cnts, Wl1, Wr1, b1.reshape(1, H),
                     Wl2, Wr2, W1, W2, W3)
    acc2 = _sc_agg2_k()(tcols, srcf, dstf, zvec)
    pab = _tc_fin(acc2, cnts, tcols, W1, W2, W3, b2.reshape(1, H),
                  c1.reshape(1, H), c2.reshape(1, H), c3.reshape(1, 1))
    outf = _sc_dec_k()(pab, am, bm)
    return outf[:EL].reshape(EL, 1)


# speedup vs baseline: 2.6708x; 1.1456x over previous
import functools

import jax
import jax.numpy as jnp
from jax import lax
from jax.experimental import pallas as pl
from jax.experimental.pallas import tpu as pltpu
from jax.experimental.pallas import tpu_sc as plsc

N = 10000
E = 320000
EL = 100000
D = 128
H = 256

NC = 2
NS = 16
L = 16
NW = NC * NS

N_PAD = 10240
R_BLK = 512
K = 128
C0 = 80
C1 = 80
T_CH = NS * (C0 + C1)
E_PAD = T_CH * K
EPT = E_PAD // NW
PER_EL = -(-EL // (NW * L)) * L
EL_PAD = NW * PER_EL

import functools as _ft


@_ft.cache
def _mesh():
    return plsc.VectorSubcoreMesh(
        core_axis_name="c", subcore_axis_name="s",
        num_cores=NC, num_subcores=NS)


def _sc_agg1_body(x_hbm, srcm, dstm, zrow, zvec, sums_out, cnts_out,
                  idx_s, idx_d, rows0, rows1, cnt_v, acc_sh,
                  sem_g, sem_s, sem_is, sem_id):
    cid = lax.axis_index("c")
    sid = lax.axis_index("s")
    wid = sid * NC + cid
    rpt = N_PAD // NS
    lo = sid * rpt
    ones16 = jnp.ones((L,), jnp.float32)
    rows = (rows0, rows1)

    def start_idx(ch, b):
        pltpu.async_copy(srcm.at[ch], idx_s.at[b], sem_is.at[b])
        pltpu.async_copy(dstm.at[ch], idx_d.at[b], sem_id.at[b])

    def wait_idx(b):
        pltpu.make_async_copy(srcm.at[0], idx_s.at[b], sem_is.at[b]).wait()
        pltpu.make_async_copy(dstm.at[0], idx_d.at[b], sem_id.at[b]).wait()

    def start_gather(bi, rb):
        pltpu.async_copy(x_hbm.at[idx_s.at[bi]], rows[rb], sem_g.at[rb])

    def wait_gather(rb):
        pltpu.make_async_copy(x_hbm.at[idx_s.at[0]], rows[rb],
                              sem_g.at[rb]).wait()

    def start_scatter(bi, rb):
        pltpu.async_copy(rows[rb], acc_sh.at[idx_d.at[bi]], sem_s.at[rb],
                         add=True)

    def wait_scatter(rb):
        pltpu.make_async_copy(rows[rb], acc_sh.at[idx_d.at[0]],
                              sem_s.at[rb]).wait()

    pltpu.sync_copy(zrow.at[pl.ds(lo, rpt)], acc_sh.at[pl.ds(lo, rpt)])
    pltpu.sync_copy(zvec, cnt_v)
    plsc.subcore_barrier()

    def run_pipe(off, n):
        for b in range(4):
            start_idx(off + b, b)
        wait_idx(0)
        start_gather(0, 0)
        wait_idx(1)
        start_gather(1, 1)

        def body(i, carry):
            for b in range(4):
                jj = i * 4 + b
                rb = b % 2
                wait_gather(rb)
                start_scatter(b, rb)
                for ll in range(K // L):
                    dv = idx_d[b, pl.ds(ll * L, L)]
                    plsc.addupdate_scatter(cnt_v, [dv], ones16)
                wait_scatter(rb)

                @pl.when(jj + 2 < n)
                def _():
                    wait_idx((b + 2) % 4)
                    start_gather((b + 2) % 4, rb)

                @pl.when(jj + 4 < n)
                def _():
                    start_idx(off + jj + 4, b)
            return carry

        lax.fori_loop(0, n // 4, body, 0)

    @pl.when(cid == 0)
    def _():
        run_pipe(sid * C0, C0)

    if C1:
        @pl.when(cid == 1)
        def _():
            run_pipe(NS * C0 + sid * C1, C1)

    plsc.subcore_barrier()
    pltpu.sync_copy(acc_sh.at[pl.ds(lo, rpt)], sums_out.at[cid, pl.ds(lo, rpt)])
    pltpu.sync_copy(cnt_v, cnts_out.at[wid])


@_ft.cache
def _sc_agg1_k():
  return functools.partial(
    pl.kernel,
    out_type=(jax.ShapeDtypeStruct((NC, N_PAD, D), jnp.float32),
              jax.ShapeDtypeStruct((NW, N_PAD), jnp.float32)),
    mesh=_mesh(),
    compiler_params=pltpu.CompilerParams(needs_layout_passes=False),
    scratch_types=[
        pltpu.VMEM((4, K), jnp.int32),
        pltpu.VMEM((4, K), jnp.int32),
        pltpu.VMEM((K, D), jnp.float32),
        pltpu.VMEM((K, D), jnp.float32),
        pltpu.VMEM((N_PAD,), jnp.float32),
        pltpu.VMEM_SHARED((N_PAD, D), jnp.float32),
        pltpu.SemaphoreType.DMA((2,)),
        pltpu.SemaphoreType.DMA((2,)),
        pltpu.SemaphoreType.DMA((4,)),
        pltpu.SemaphoreType.DMA((4,)),
    ],
)(_sc_agg1_body)


def _sc_agg2_body(tcols, srcf, dstf, zvec, acc2_out,
                  src_v, dst_v, ta_v, tb_v, acca_v, accb_v):
    cid = lax.axis_index("c")
    sid = lax.axis_index("s")
    wid = sid * NC + cid
    pltpu.sync_copy(zvec, acca_v)
    pltpu.sync_copy(zvec, accb_v)
    pltpu.sync_copy(tcols.at[2], ta_v)
    pltpu.sync_copy(tcols.at[3], tb_v)
    pltpu.sync_copy(srcf.at[wid], src_v)
    pltpu.sync_copy(dstf.at[wid], dst_v)

    def body(j, carry):
        off = j * L
        sv = src_v[pl.ds(off, L)]
        dv = dst_v[pl.ds(off, L)]
        va = plsc.load_gather(ta_v, [sv])
        vb = plsc.load_gather(tb_v, [sv])
        plsc.addupdate_scatter(acca_v, [dv], va)
        plsc.addupdate_scatter(accb_v, [dv], vb)
        return carry

    lax.fori_loop(0, EPT // L, body, 0)
    pltpu.sync_copy(acca_v, acc2_out.at[wid, 0])
    pltpu.sync_copy(accb_v, acc2_out.at[wid, 1])


@_ft.cache
def _sc_agg2_k():
  return functools.partial(
    pl.kernel,
    out_type=jax.ShapeDtypeStruct((NW, 2, N_PAD), jnp.float32),
    mesh=_mesh(),
    compiler_params=pltpu.CompilerParams(needs_layout_passes=False),
    scratch_types=[
        pltpu.VMEM((EPT,), jnp.int32),
        pltpu.VMEM((EPT,), jnp.int32),
        pltpu.VMEM((N_PAD,), jnp.float32),
        pltpu.VMEM((N_PAD,), jnp.float32),
        pltpu.VMEM((N_PAD,), jnp.float32),
        pltpu.VMEM((N_PAD,), jnp.float32),
    ],
)(_sc_agg2_body)


def _sc_dec_body(pab_hbm, am, bm, out_hbm, pa_v, pb_v, ai_v, bi_v, out_v):
    cid = lax.axis_index("c")
    sid = lax.axis_index("s")
    wid = sid * NC + cid
    base = wid * PER_EL
    pltpu.sync_copy(pab_hbm.at[0], pa_v)
    pltpu.sync_copy(pab_hbm.at[1], pb_v)
    pltpu.sync_copy(am.at[wid], ai_v)
    pltpu.sync_copy(bm.at[wid], bi_v)

    def body(j, carry):
        off = j * L
        ia = ai_v[pl.ds(off, L)]
        ib = bi_v[pl.ds(off, L)]
        va = plsc.load_gather(pa_v, [ia])
        vb = plsc.load_gather(pb_v, [ib])
        out_v[pl.ds(off, L)] = va + vb
        return carry

    lax.fori_loop(0, PER_EL // L, body, 0)
    pltpu.sync_copy(out_v, out_hbm.at[pl.ds(base, PER_EL)])


@_ft.cache
def _sc_dec_k():
  return functools.partial(
    pl.kernel,
    out_type=jax.ShapeDtypeStruct((EL_PAD,), jnp.float32),
    mesh=_mesh(),
    compiler_params=pltpu.CompilerParams(needs_layout_passes=False),
    scratch_types=[
        pltpu.VMEM((N_PAD,), jnp.float32),
        pltpu.VMEM((N_PAD,), jnp.float32),
        pltpu.VMEM((PER_EL,), jnp.int32),
        pltpu.VMEM((PER_EL,), jnp.int32),
        pltpu.VMEM((PER_EL,), jnp.float32),
    ],
)(_sc_dec_body)


def _tc_main_body(x_ref, s_ref, c_ref, wl1_ref, wr1_ref, b1_ref,
                  wl2_ref, wr2_ref, w1_ref, w2_ref, w3_ref, t_ref, v_ref):
    @pl.when(pl.program_id(0) == 0)
    def _():
        w23 = jnp.dot(w2_ref[...], w3_ref[...],
                      preferred_element_type=jnp.float32,
                      precision=jax.lax.Precision.HIGHEST)
        wfull = jnp.dot(w1_ref[...], w23, preferred_element_type=jnp.float32,
                        precision=jax.lax.Precision.HIGHEST)
        wa = wfull[:H]
        wb = wfull[H:]
        v_ref[...] = jnp.concatenate(
            [jnp.dot(wr2_ref[...], wa, preferred_element_type=jnp.float32,
                     precision=jax.lax.Precision.HIGHEST),
             jnp.dot(wr2_ref[...], wb, preferred_element_type=jnp.float32,
                     precision=jax.lax.Precision.HIGHEST),
             jnp.dot(wl2_ref[...], wa, preferred_element_type=jnp.float32,
                     precision=jax.lax.Precision.HIGHEST),
             jnp.dot(wl2_ref[...], wb, preferred_element_type=jnp.float32,
                     precision=jax.lax.Precision.HIGHEST)],
            axis=1)

    s = s_ref[0] + s_ref[1]
    cnt = jnp.sum(c_ref[...], axis=0)[:, None]
    mean = s / jnp.maximum(cnt, 1.0)
    z1 = jnp.maximum(
        jnp.dot(mean, wl1_ref[...], preferred_element_type=jnp.float32,
                precision=jax.lax.Precision.HIGHEST)
        + jnp.dot(x_ref[...], wr1_ref[...],
                  preferred_element_type=jnp.float32,
                  precision=jax.lax.Precision.HIGHEST)
        + b1_ref[...], 0.0)
    t_ref[...] = jnp.dot(z1, v_ref[...], preferred_element_type=jnp.float32,
                         precision=jax.lax.Precision.HIGHEST).T


def _tc_main(x_pad, sums, cnts, Wl1, Wr1, b1, Wl2, Wr2, W1, W2, W3):
    grid = (N_PAD // R_BLK,)
    return pl.pallas_call(
        _tc_main_body,
        grid=grid,
        in_specs=[
            pl.BlockSpec((R_BLK, D), lambda i: (i, 0)),
            pl.BlockSpec((NC, R_BLK, D), lambda i: (0, i, 0)),
            pl.BlockSpec((NW, R_BLK), lambda i: (0, i)),
            pl.BlockSpec((D, H), lambda i: (0, 0)),
            pl.BlockSpec((D, H), lambda i: (0, 0)),
            pl.BlockSpec((1, H), lambda i: (0, 0)),
            pl.BlockSpec((H, H), lambda i: (0, 0)),
            pl.BlockSpec((H, H), lambda i: (0, 0)),
            pl.BlockSpec((2 * H, H), lambda i: (0, 0)),
            pl.BlockSpec((H, H), lambda i: (0, 0)),
            pl.BlockSpec((H, 1), lambda i: (0, 0)),
        ],
        out_specs=pl.BlockSpec((4, R_BLK), lambda i: (0, i)),
        out_shape=jax.ShapeDtypeStruct((4, N_PAD), jnp.float32),
        scratch_shapes=[pltpu.VMEM((H, 4), jnp.float32)],
    )(x_pad, sums, cnts, Wl1, Wr1, b1, Wl2, Wr2, W1, W2, W3)


def _tc_fin_body(acc2_ref, cnts_ref, t_ref, w1_ref, w2_ref, w3_ref,
                 b2_ref, c1_ref, c2_ref, c3_ref, pab_ref):
    w23 = jnp.dot(w2_ref[...], w3_ref[...], preferred_element_type=jnp.float32, precision=jax.lax.Precision.HIGHEST)
    wfull = jnp.dot(w1_ref[...], w23, preferred_element_type=jnp.float32, precision=jax.lax.Precision.HIGHEST)
    wa = wfull[:H, 0]
    wb = wfull[H:, 0]
    cfull = (jnp.sum(c1_ref[0] * w23[:, 0]) + jnp.sum(c2_ref[0] * w3_ref[:, 0])
             + c3_ref[0, 0])
    ba = jnp.sum(b2_ref[0] * wa)
    bb = jnp.sum(b2_ref[0] * wb)
    cnt = jnp.maximum(jnp.sum(cnts_ref[...], axis=0), 1.0)
    m_a = jnp.sum(acc2_ref[:, 0, :], axis=0) / cnt
    m_b = jnp.sum(acc2_ref[:, 1, :], axis=0) / cnt
    pab_ref[0, :] = m_a + t_ref[0, :] + ba + cfull
    pab_ref[1, :] = m_b + t_ref[1, :] + bb


def _tc_fin(acc2, cnts, tcols, W1, W2, W3, b2, c1, c2, c3):
    return pl.pallas_call(
        _tc_fin_body,
        out_shape=jax.ShapeDtypeStruct((2, N_PAD), jnp.float32),
    )(acc2, cnts, tcols, W1, W2, W3, b2, c1, c2, c3)


def kernel(x, edge_index, edge_label_index, Wl1, Wr1, b1, Wl2, Wr2, b2,
           W1, c1, W2, c2, W3, c3):
    f32 = jnp.float32
    i32 = jnp.int32
    src = edge_index[0].astype(i32)
    dst = edge_index[1].astype(i32)
    pad_src = jnp.arange(E_PAD - E, dtype=i32) % N
    pad_dst = N + jnp.arange(E_PAD - E, dtype=i32) % (N_PAD - N)
    src_pad = jnp.concatenate([src, pad_src])
    dst_pad = jnp.concatenate([dst, pad_dst])
    srcm = src_pad.reshape(T_CH, K)
    dstm = dst_pad.reshape(T_CH, K)
    srcf = src_pad.reshape(NW, EPT)
    dstf = dst_pad.reshape(NW, EPT)
    am = jnp.concatenate(
        [edge_label_index[0].astype(i32),
         jnp.zeros((EL_PAD - EL,), i32)]).reshape(NW, PER_EL)
    bm = jnp.concatenate(
        [edge_label_index[1].astype(i32),
         jnp.zeros((EL_PAD - EL,), i32)]).reshape(NW, PER_EL)
    x_pad = jnp.zeros((N_PAD, D), f32).at[:N].set(x)
    zrow = jnp.zeros((N_PAD, D), f32)
    zvec = jnp.zeros((N_PAD,), f32)

    sums, cnts = _sc_agg1_k()(x_pad, srcm, dstm, zrow, zvec)
    tcols = _tc_main(x_pad, sums, cnts, Wl1, Wr1, b1.reshape(1, H),
                     Wl2, Wr2, W1, W2, W3)
    acc2 = _sc_agg2_k()(tcols, srcf, dstf, zvec)
    pab = _tc_fin(acc2, cnts, tcols, W1, W2, W3, b2.reshape(1, H),
                  c1.reshape(1, H), c2.reshape(1, H), c3.reshape(1, 1))
    outf = _sc_dec_k()(pab, am, bm)
    return outf[:EL].reshape(EL, 1)
